# Initial kernel scaffold; baseline (speedup 1.0000x reference)
import jax
import jax.numpy as jnp
from jax.experimental import pallas as pl


def kernel(rel_height, rel_width):
    raise NotImplementedError("write your pallas kernel here")


# speedup vs baseline: 221.0059x; 221.0059x over previous
import jax
import jax.numpy as jnp
from jax import lax
from jax.experimental import pallas as pl

_H, _W, _NH = 32, 32, 16
_L = _H * _W
_KH = 2 * _H - 1
_KW = 2 * _W - 1


def _bias_kernel(rh_ref, rw_ref, out_ref):
    u = rh_ref[0, 0, :]
    v = rw_ref[0, 0, :]

    r = lax.broadcasted_iota(jnp.int32, (_H, _H, _KH), 0)
    rp = lax.broadcasted_iota(jnp.int32, (_H, _H, _KH), 1)
    k = lax.broadcasted_iota(jnp.int32, (_H, _H, _KH), 2)
    oh = (r - rp + (_H - 1) == k).astype(jnp.float32)
    A = jnp.sum(oh * u[None, None, :], axis=-1)
    B = jnp.sum(oh * v[None, None, :], axis=-1)

    i = lax.broadcasted_iota(jnp.int32, (_L, _H), 0)
    g = lax.broadcasted_iota(jnp.int32, (_L, _H), 1)
    E = (i // _W == g).astype(jnp.float32)
    F = (i % _W == g).astype(jnp.float32)
    g2 = lax.broadcasted_iota(jnp.int32, (_H, _L), 0)
    j = lax.broadcasted_iota(jnp.int32, (_H, _L), 1)
    Et = (j // _W == g2).astype(jnp.float32)
    Ft = (j % _W == g2).astype(jnp.float32)

    xa = jnp.dot(E, A, preferred_element_type=jnp.float32)
    xb = jnp.dot(F, B, preferred_element_type=jnp.float32)
    out_ref[0, :, :] = (
        jnp.dot(xa, Et, preferred_element_type=jnp.float32)
        + jnp.dot(xb, Ft, preferred_element_type=jnp.float32)
    )


def kernel(rel_height, rel_width):
    rh = rel_height.T.reshape(_NH, 1, _KH)
    rw = rel_width.T.reshape(_NH, 1, _KW)
    return pl.pallas_call(
        _bias_kernel,
        grid=(_NH,),
        in_specs=[
            pl.BlockSpec((1, 1, _KH), lambda h: (h, 0, 0)),
            pl.BlockSpec((1, 1, _KW), lambda h: (h, 0, 0)),
        ],
        out_specs=pl.BlockSpec((1, _L, _L), lambda h: (h, 0, 0)),
        out_shape=jax.ShapeDtypeStruct((_NH, _L, _L), jnp.float32),
    )(rh, rw)
